# baseline (speedup 1.0000x reference)
import jax
import jax.numpy as jnp
from jax import lax
from jax.experimental import pallas as pl
from jax.experimental.pallas import tpu as pltpu
from jax.experimental.pallas import tpu_sc as plsc

D = 128
NROWS = 100
N = 100000
NC, NS = 2, 16
NW = NC * NS
BPW = N // NW
CHUNK = 125
CPW = BPW // CHUNK
NBUF = 7
NLANE = 16
GRP = (CHUNK // NLANE) * NLANE
TGRP = CHUNK - NLANE
ISTAGE = (BPW // 8 + 2) * 8


def _emb_body(z_hbm, tab_hbm, out_hbm, tab_v, idx_v, stage, wsems):
    wid = lax.axis_index("s") * NC + lax.axis_index("c")
    base = wid * BPW
    astart = lax.min((base // 8) * 8, N - ISTAGE)
    s = base - astart
    pltpu.sync_copy(tab_hbm, tab_v)
    pltpu.sync_copy(z_hbm.at[pl.ds(astart, ISTAGE)], idx_v)

    CW = CHUNK * D

    def write(j):
        boff = (j % NBUF) * CW
        return pltpu.make_async_copy(
            stage.at[pl.ds(boff, CW)],
            out_hbm.at[pl.ds((base + j * CHUNK) * D, CW)],
            wsems.at[j % NBUF])

    def chunk(j, carry):
        boff = (j % NBUF) * CW

        @pl.when(j >= NBUF)
        def _():
            write(j - NBUF).wait()

        def group(i):
            zv = idx_v[pl.ds(s + j * CHUNK + i, NLANE)]
            for k in range(NLANE):
                off = zv[k] * D
                dst = boff + i * D + k * D
                row = [tab_v[pl.ds(off + c * NLANE, NLANE)]
                       for c in range(D // NLANE)]
                for c in range(D // NLANE):
                    stage[pl.ds(dst + c * NLANE, NLANE)] = row[c]

        @plsc.parallel_loop(0, GRP + NLANE, step=NLANE)
        def _group(i):
            group(lax.min(i, TGRP))

        write(j).start()
        return carry

    lax.fori_loop(0, CPW, chunk, 0)
    for j in range(CPW - NBUF, CPW):
        write(j).wait()


@jax.jit
def _emb(z1d, tab_flat):
    f = pl.kernel(
        _emb_body,
        out_type=jax.ShapeDtypeStruct((N * D,), jnp.float32),
        mesh=plsc.VectorSubcoreMesh(core_axis_name="c", subcore_axis_name="s"),
        scratch_types=[
            pltpu.VMEM((NROWS * D,), jnp.float32),
            pltpu.VMEM((ISTAGE,), jnp.int32),
            pltpu.VMEM((NBUF * CHUNK * D,), jnp.float32),
            pltpu.SemaphoreType.DMA((NBUF,)),
        ],
    )
    return f(z1d, tab_flat)


def kernel(Z, emb_weight):
    out = _emb(Z.astype(jnp.int32), emb_weight.reshape(-1))
    return out.reshape(N, D)
